# baseline (speedup 1.0000x reference)
import jax
import jax.numpy as jnp
from jax.experimental import pallas as pl
from jax.experimental.pallas import tpu as pltpu

_N = 100000
_D = 768
_BN = 1000
_G = _N // _BN


def _body(pv_ref, s_ref, w_ref, b_ref, off_ref, emb_ref, out_ref, acc_ref, ps_ref):
    i = pl.program_id(0)

    @pl.when(i == 0)
    def _():
        acc_ref[...] = jnp.zeros_like(acc_ref)
        pv_all = pv_ref[...]
        ps_ref[0] = jnp.sum(pv_all * s_ref[...])
        ps_ref[1] = jnp.sum(pv_all)

    pv = pv_ref[pl.ds(i, 1), :]
    acc_ref[...] += jnp.dot(pv, emb_ref[...], preferred_element_type=jnp.float32)

    @pl.when(i == _G - 1)
    def _():
        total = jnp.dot(acc_ref[...], w_ref[...], preferred_element_type=jnp.float32)
        out_ref[...] = total + ps_ref[0] + ps_ref[1] * (b_ref[0] + off_ref[0])


def kernel(phrase_vector, score, W, b, offset, emb_table):
    return pl.pallas_call(
        _body,
        grid=(_G,),
        in_specs=[
            pl.BlockSpec((_G, _BN), lambda i: (0, 0)),
            pl.BlockSpec((_G, _BN), lambda i: (0, 0)),
            pl.BlockSpec((_D, 1), lambda i: (0, 0)),
            pl.BlockSpec(memory_space=pltpu.SMEM),
            pl.BlockSpec(memory_space=pltpu.SMEM),
            pl.BlockSpec((_BN, _D), lambda i: (i, 0)),
        ],
        out_specs=pl.BlockSpec((1, 1), lambda i: (0, 0)),
        out_shape=jax.ShapeDtypeStruct((1, 1), jnp.float32),
        scratch_shapes=[
            pltpu.VMEM((1, _D), jnp.float32),
            pltpu.SMEM((2,), jnp.float32),
        ],
    )(phrase_vector.reshape(_G, _BN), score.reshape(_G, _BN), W, b, offset, emb_table)


# speedup vs baseline: 1.2730x; 1.2730x over previous
import jax
import jax.numpy as jnp
from jax.experimental import pallas as pl
from jax.experimental.pallas import tpu as pltpu

_N = 100000
_D = 768
_BN = 4000
_G = _N // _BN


def _body(pv_ref, s_ref, w_ref, b_ref, off_ref, emb_ref, out_ref, acc_ref, ps_ref):
    i = pl.program_id(0)

    @pl.when(i == 0)
    def _():
        acc_ref[...] = jnp.zeros_like(acc_ref)
        pv_all = pv_ref[...]
        ps_ref[0] = jnp.sum(pv_all * s_ref[...])
        ps_ref[1] = jnp.sum(pv_all)

    pv = pv_ref[pl.ds(i, 1), :]
    acc_ref[...] += jnp.dot(pv, emb_ref[...], preferred_element_type=jnp.float32)

    @pl.when(i == _G - 1)
    def _():
        total = jnp.dot(acc_ref[...], w_ref[...], preferred_element_type=jnp.float32)
        out_ref[...] = total + ps_ref[0] + ps_ref[1] * (b_ref[0] + off_ref[0])


def kernel(phrase_vector, score, W, b, offset, emb_table):
    return pl.pallas_call(
        _body,
        grid=(_G,),
        in_specs=[
            pl.BlockSpec((_G, _BN), lambda i: (0, 0)),
            pl.BlockSpec((_G, _BN), lambda i: (0, 0)),
            pl.BlockSpec((_D, 1), lambda i: (0, 0)),
            pl.BlockSpec(memory_space=pltpu.SMEM),
            pl.BlockSpec(memory_space=pltpu.SMEM),
            pl.BlockSpec((_BN, _D), lambda i: (i, 0)),
        ],
        out_specs=pl.BlockSpec((1, 1), lambda i: (0, 0)),
        out_shape=jax.ShapeDtypeStruct((1, 1), jnp.float32),
        scratch_shapes=[
            pltpu.VMEM((1, _D), jnp.float32),
            pltpu.SMEM((2,), jnp.float32),
        ],
    )(phrase_vector.reshape(_G, _BN), score.reshape(_G, _BN), W, b, offset, emb_table)
